# Initial kernel scaffold; baseline (speedup 1.0000x reference)
import jax
import jax.numpy as jnp
from jax.experimental import pallas as pl


def kernel(attn_bias, spatial_pos, attn_edge_type, spatial_W, edge_W, token_W):
    raise NotImplementedError("write your pallas kernel here")


# speedup vs baseline: 13.0210x; 13.0210x over previous
import jax
import jax.numpy as jnp
from jax import lax
from jax.experimental import pallas as pl
from jax.experimental.pallas import tpu as pltpu
from jax.experimental.pallas import tpu_sc as plsc


def _bias_encoder_sc(B, N, H, FE):
    NP1 = N + 1
    W = H // 2
    JB = N // 16
    EOFF = 513 * W

    mesh = plsc.VectorSubcoreMesh(core_axis_name="c", subcore_axis_name="s")

    def tec(ab, sp, et, tw, tok, tokr, out, tv, tile, spv, etv, abv, tokv,
            tokrv):
        cid = lax.axis_index("c")
        sid = lax.axis_index("s")
        wid = sid * 2 + cid
        pltpu.sync_copy(tw, tv)
        pltpu.sync_copy(tok, tokv)
        pltpu.sync_copy(tokr, tokrv)
        i16 = lax.iota(jnp.int32, 16)
        i4 = i16 * FE
        zero16 = jnp.zeros((16,), jnp.int32)
        lane0 = i16 == 0

        def emit_row(b, orow, with_emb):
            av = abv[pl.ds(0, 16)]
            ab0 = jnp.sum(jnp.where(lane0, av, 0.0)) * 2.0
            plsc.store_scatter(tile, [i16, zero16], tokv[pl.ds(0, 16)] + ab0)
            plsc.store_scatter(tile, [i16 + 16, zero16],
                               tokv[pl.ds(16, 16)] + ab0)
            for jb in range(JB):
                ab2 = plsc.load_gather(abv, [i16 + (16 * jb + 1)]) * 2.0
                if with_emb:
                    sp16 = spv[pl.ds(16 * jb, 16)]
                    spf = sp16 * W
                    ef = [plsc.load_gather(etv, [i4 + (16 * FE * jb + f)]) * W
                          + EOFF for f in range(FE)]
                for w in range(W):
                    if with_emb:
                        acc = plsc.bitcast(plsc.load_gather(tv, [spf + w]),
                                           jnp.bfloat16)
                        for e in ef:
                            acc = acc + plsc.bitcast(
                                plsc.load_gather(tv, [e + w]), jnp.bfloat16)
                        si = plsc.bitcast(acc, jnp.int32)
                        hE = plsc.bitcast(si << 16, jnp.float32) + ab2
                        hO = plsc.bitcast(si & jnp.int32(-65536),
                                          jnp.float32) + ab2
                    else:
                        hE = ab2 + tokrv[2 * w, :]
                        hO = ab2 + tokrv[2 * w + 1, :]
                    rE = jnp.full((16,), 2 * w, jnp.int32)
                    rO = jnp.full((16,), 2 * w + 1, jnp.int32)
                    cidx = i16 + (16 * jb + 1)
                    plsc.store_scatter(tile, [rE, cidx], hE)
                    plsc.store_scatter(tile, [rO, cidx], hO)
            pltpu.sync_copy(tile, out.at[b, :, orow, :])

        for bb in range(2):
            b = wid * 2 + bb

            def body(i, c):
                pltpu.sync_copy(sp.at[b, i, :], spv)
                pltpu.sync_copy(et.at[b, i, :], etv)
                pltpu.sync_copy(ab.at[b, i + 1, :], abv)
                emit_row(b, i + 1, True)
                return c

            lax.fori_loop(0, N, body, 0)
            pltpu.sync_copy(ab.at[b, 0, :], abv)
            emit_row(b, 0, False)

    return pl.kernel(
        tec,
        out_type=jax.ShapeDtypeStruct((B, H, NP1, NP1), jnp.float32),
        mesh=mesh,
        compiler_params=pltpu.CompilerParams(use_tc_tiling_on_sc=False,
                                             needs_layout_passes=False),
        scratch_types=[
            pltpu.VMEM((1027 * W,), jnp.int32),
            pltpu.VMEM((H, NP1), jnp.float32),
            pltpu.VMEM((N,), jnp.int32),
            pltpu.VMEM((N * FE,), jnp.int32),
            pltpu.VMEM((NP1,), jnp.float32),
            pltpu.VMEM((H,), jnp.float32),
            pltpu.VMEM((H, 16), jnp.float32),
        ],
    )


def kernel(attn_bias, spatial_pos, attn_edge_type, spatial_W, edge_W, token_W):
    B, NP1, _ = attn_bias.shape
    N = NP1 - 1
    H = spatial_W.shape[1]
    FE = attn_edge_type.shape[-1]

    tb = jnp.concatenate([spatial_W, edge_W / FE], axis=0).astype(jnp.bfloat16)
    tw = lax.bitcast_convert_type(tb.reshape(-1, H // 2, 2),
                                  jnp.int32).reshape(-1)
    tok = token_W.reshape(H).astype(jnp.float32)
    tokr = jnp.tile(tok.reshape(H, 1), (1, 16))
    sp = spatial_pos.astype(jnp.int32)
    et = attn_edge_type.reshape(B, N, N * FE).astype(jnp.int32)

    run = _bias_encoder_sc(B, N, H, FE)
    return run(attn_bias, sp, et, tw, tok, tokr)


# speedup vs baseline: 17.7755x; 1.3651x over previous
import jax
import jax.numpy as jnp
from jax import lax
from jax.experimental import pallas as pl
from jax.experimental.pallas import tpu as pltpu
from jax.experimental.pallas import tpu_sc as plsc


def _bias_encoder_sc(B, N, H, FE):
    NP1 = N + 1
    PC = 136
    W = H // 2
    JB = N // 16
    EOFF = 513 * W

    mesh = plsc.VectorSubcoreMesh(core_axis_name="c", subcore_axis_name="s")

    def tec(ab, sp, et, tw, tok, tokr, out,
            tv, t0, t1, spa, spb, eta, etb, aba, abb, tokv, tokrv,
            sia, sib, st0, st1):
        cid = lax.axis_index("c")
        sid = lax.axis_index("s")
        wid = sid * 2 + cid
        pltpu.sync_copy(tw, tv)
        pltpu.sync_copy(tok, tokv)
        pltpu.sync_copy(tokr, tokrv)
        i16 = lax.iota(jnp.int32, 16)
        i4 = i16 * FE
        zero16 = jnp.zeros((16,), jnp.int32)
        lane0 = i16 == 0

        def start_in(b, i, spx, etx, abx, sem):
            pltpu.async_copy(sp.at[b, i, :], spx, sem)
            pltpu.async_copy(et.at[b, i, :], etx, sem)
            pltpu.async_copy(ab.at[b, i + 1, :], abx, sem)

        def wait_in(b, i, spx, etx, abx, sem):
            pltpu.make_async_copy(sp.at[b, i, :], spx, sem).wait()
            pltpu.make_async_copy(et.at[b, i, :], etx, sem).wait()
            pltpu.make_async_copy(ab.at[b, i + 1, :], abx, sem).wait()

        def emit_main(spx, etx, abx, tile):
            av = abx[pl.ds(0, 16)]
            ab0 = jnp.sum(jnp.where(lane0, av, 0.0)) * 2.0
            plsc.store_scatter(tile, [i16, zero16], tokv[pl.ds(0, 16)] + ab0)
            plsc.store_scatter(tile, [i16 + 16, zero16],
                               tokv[pl.ds(16, 16)] + ab0)

            def jb_body(jb, c):
                ab2 = plsc.load_gather(abx, [i16 + (16 * jb + 1)]) * 2.0
                sp16 = plsc.load_gather(spx, [i16 + 16 * jb])
                spf = sp16 * W
                ef = [plsc.load_gather(etx, [i4 + (16 * FE * jb + f)]) * W
                      + EOFF for f in range(FE)]
                cidx = i16 + (16 * jb + 1)
                for w in range(W):
                    acc = plsc.bitcast(plsc.load_gather(tv, [spf + w]),
                                       jnp.bfloat16)
                    for e in ef:
                        acc = acc + plsc.bitcast(
                            plsc.load_gather(tv, [e + w]), jnp.bfloat16)
                    si = plsc.bitcast(acc, jnp.int32)
                    hE = plsc.bitcast(si << 16, jnp.float32) + ab2
                    hO = plsc.bitcast(si & jnp.int32(-65536), jnp.float32) + ab2
                    plsc.store_scatter(
                        tile, [jnp.full((16,), 2 * w, jnp.int32), cidx], hE)
                    plsc.store_scatter(
                        tile, [jnp.full((16,), 2 * w + 1, jnp.int32), cidx], hO)
                return c

            lax.fori_loop(0, JB, jb_body, 0)

        def emit_row0(abx, tile):
            def jb_body(jb, c):
                cidx = i16 + 16 * jb
                msk = cidx < NP1
                ab2 = plsc.load_gather(abx, [jnp.minimum(cidx, PC - 1)]) * 2.0
                for w in range(H):
                    plsc.store_scatter(
                        tile, [jnp.full((16,), w, jnp.int32), cidx],
                        ab2 + tokrv[w, :], mask=msk)
                return c

            lax.fori_loop(0, (PC + 15) // 16, jb_body, 0)

        for bb in range(2):
            b = wid * 2 + bb
            start_in(b, 0, spa, eta, aba, sia)

            def pair(q, c):
                i0 = 2 * q
                wait_in(b, i0, spa, eta, aba, sia)
                start_in(b, i0 + 1, spb, etb, abb, sib)

                @pl.when(q > 0)
                def _():
                    pltpu.make_async_copy(t0, out.at[b, :, i0 + 1, :],
                                          st0).wait()

                emit_main(spa, eta, aba, t0)
                pltpu.async_copy(t0, out.at[b, :, i0 + 1, :], st0)

                wait_in(b, i0 + 1, spb, etb, abb, sib)

                @pl.when(q < N // 2 - 1)
                def _():
                    start_in(b, i0 + 2, spa, eta, aba, sia)

                @pl.when(q > 0)
                def _():
                    pltpu.make_async_copy(t1, out.at[b, :, i0 + 2, :],
                                          st1).wait()

                emit_main(spb, etb, abb, t1)
                pltpu.async_copy(t1, out.at[b, :, i0 + 2, :], st1)
                return c

            lax.fori_loop(0, N // 2, pair, 0)
            pltpu.make_async_copy(t0, out.at[b, :, N - 1, :], st0).wait()
            pltpu.make_async_copy(t1, out.at[b, :, N, :], st1).wait()
            pltpu.sync_copy(ab.at[b, 0, :], aba)
            emit_row0(aba, t0)
            pltpu.sync_copy(t0, out.at[b, :, 0, :])

    return pl.kernel(
        tec,
        out_type=jax.ShapeDtypeStruct((B, H, NP1, PC), jnp.float32),
        mesh=mesh,
        compiler_params=pltpu.CompilerParams(use_tc_tiling_on_sc=False,
                                             needs_layout_passes=False),
        scratch_types=[
            pltpu.VMEM((1027 * W,), jnp.int32),
            pltpu.VMEM((H, PC), jnp.float32),
            pltpu.VMEM((H, PC), jnp.float32),
            pltpu.VMEM((N,), jnp.int32),
            pltpu.VMEM((N,), jnp.int32),
            pltpu.VMEM((N * FE,), jnp.int32),
            pltpu.VMEM((N * FE,), jnp.int32),
            pltpu.VMEM((PC,), jnp.float32),
            pltpu.VMEM((PC,), jnp.float32),
            pltpu.VMEM((H,), jnp.float32),
            pltpu.VMEM((H, 16), jnp.float32),
            pltpu.SemaphoreType.DMA,
            pltpu.SemaphoreType.DMA,
            pltpu.SemaphoreType.DMA,
            pltpu.SemaphoreType.DMA,
        ],
    )


def kernel(attn_bias, spatial_pos, attn_edge_type, spatial_W, edge_W, token_W):
    B, NP1, _ = attn_bias.shape
    N = NP1 - 1
    H = spatial_W.shape[1]
    FE = attn_edge_type.shape[-1]
    PC = 136

    tb = jnp.concatenate([spatial_W, edge_W / FE], axis=0).astype(jnp.bfloat16)
    tw = lax.bitcast_convert_type(tb.reshape(-1, H // 2, 2),
                                  jnp.int32).reshape(-1)
    tok = token_W.reshape(H).astype(jnp.float32)
    tokr = jnp.tile(tok.reshape(H, 1), (1, 16))
    sp = spatial_pos.astype(jnp.int32)
    et = attn_edge_type.reshape(B, N, N * FE).astype(jnp.int32)
    abp = jnp.pad(attn_bias, ((0, 0), (0, 0), (0, PC - NP1)))

    run = _bias_encoder_sc(B, N, H, FE)
    outp = run(abp, sp, et, tw, tok, tokr)
    return outp[:, :, :, :NP1]


# speedup vs baseline: 28.9885x; 1.6308x over previous
import jax
import jax.numpy as jnp
from jax import lax
from jax.experimental import pallas as pl
from jax.experimental.pallas import tpu as pltpu
from jax.experimental.pallas import tpu_sc as plsc


def _bias_encoder_sc(B, N, H, FE):
    NP1 = N + 1
    PC = 136
    W = H // 2
    JB = N // 16
    EOFF = 513 * W

    mesh = plsc.VectorSubcoreMesh(core_axis_name="c", subcore_axis_name="s")

    def tec(ab, sp, et, tw, tok, tokr, out,
            tv, t0, t1, spa, spb, eta, etb, aba, abb, tokv, tokrv,
            sia, sib, st0, st1):
        cid = lax.axis_index("c")
        sid = lax.axis_index("s")
        wid = sid * 2 + cid
        pltpu.sync_copy(tw, tv)
        pltpu.sync_copy(tok, tokv)
        pltpu.sync_copy(tokr, tokrv)
        i16 = lax.iota(jnp.int32, 16)
        i4 = i16 * FE
        zero16 = jnp.zeros((16,), jnp.int32)
        lane0 = i16 == 0

        def start_in(b, i, spx, etx, abx, sem):
            pltpu.async_copy(sp.at[b, i, :], spx, sem)
            pltpu.async_copy(et.at[b, i, :], etx, sem)
            pltpu.async_copy(ab.at[b, i + 1, :], abx, sem)

        def wait_in(b, i, spx, etx, abx, sem):
            pltpu.make_async_copy(sp.at[b, i, :], spx, sem).wait()
            pltpu.make_async_copy(et.at[b, i, :], etx, sem).wait()
            pltpu.make_async_copy(ab.at[b, i + 1, :], abx, sem).wait()

        def emit_main(spx, etx, abx, tile):
            av = abx[pl.ds(0, 16)]
            ab0 = jnp.sum(jnp.where(lane0, av, 0.0)) * 2.0
            plsc.store_scatter(tile, [i16, zero16], tokv[pl.ds(0, 16)] + ab0)
            plsc.store_scatter(tile, [i16 + 16, zero16],
                               tokv[pl.ds(16, 16)] + ab0)

            def jb_body(jb, c):
                ab2 = plsc.load_gather(abx, [i16 + (16 * jb + 1)]) * 2.0
                sp16 = plsc.load_gather(spx, [i16 + 16 * jb])
                spf = sp16 * W
                ef = [plsc.load_gather(etx, [i4 + (16 * FE * jb + f)]) * W
                      + EOFF for f in range(FE)]
                cidx = i16 + (16 * jb + 1)
                for w in range(W):
                    wv = (i16 + w) & (W - 1)
                    acc = plsc.bitcast(plsc.load_gather(tv, [spf + wv]),
                                       jnp.bfloat16)
                    for e in ef:
                        acc = acc + plsc.bitcast(
                            plsc.load_gather(tv, [e + wv]), jnp.bfloat16)
                    si = plsc.bitcast(acc, jnp.int32)
                    hE = plsc.bitcast(si << 16, jnp.float32) + ab2
                    hO = plsc.bitcast(si & jnp.int32(-65536), jnp.float32) + ab2
                    rE = wv + wv
                    plsc.store_scatter(tile, [rE, cidx], hE)
                    plsc.store_scatter(tile, [rE + 1, cidx], hO)
                return c

            lax.fori_loop(0, JB, jb_body, 0)

        def emit_row0(abx, tile):
            def jb_body(jb, c):
                cidx = i16 + 16 * jb
                msk = cidx < NP1
                ab2 = plsc.load_gather(abx, [jnp.minimum(cidx, PC - 1)]) * 2.0
                for w in range(H):
                    plsc.store_scatter(
                        tile, [jnp.full((16,), w, jnp.int32), cidx],
                        ab2 + tokrv[w, :], mask=msk)
                return c

            lax.fori_loop(0, (PC + 15) // 16, jb_body, 0)

        for bb in range(2):
            b = wid * 2 + bb
            start_in(b, 0, spa, eta, aba, sia)

            def pair(q, c):
                i0 = 2 * q
                wait_in(b, i0, spa, eta, aba, sia)
                start_in(b, i0 + 1, spb, etb, abb, sib)

                @pl.when(q > 0)
                def _():
                    pltpu.make_async_copy(t0, out.at[b, :, i0 + 1, :],
                                          st0).wait()

                emit_main(spa, eta, aba, t0)
                pltpu.async_copy(t0, out.at[b, :, i0 + 1, :], st0)

                wait_in(b, i0 + 1, spb, etb, abb, sib)

                @pl.when(q < N // 2 - 1)
                def _():
                    start_in(b, i0 + 2, spa, eta, aba, sia)

                @pl.when(q > 0)
                def _():
                    pltpu.make_async_copy(t1, out.at[b, :, i0 + 2, :],
                                          st1).wait()

                emit_main(spb, etb, abb, t1)
                pltpu.async_copy(t1, out.at[b, :, i0 + 2, :], st1)
                return c

            lax.fori_loop(0, N // 2, pair, 0)
            pltpu.make_async_copy(t0, out.at[b, :, N - 1, :], st0).wait()
            pltpu.make_async_copy(t1, out.at[b, :, N, :], st1).wait()
            pltpu.sync_copy(ab.at[b, 0, :], aba)
            emit_row0(aba, t0)
            pltpu.sync_copy(t0, out.at[b, :, 0, :])

    return pl.kernel(
        tec,
        out_type=jax.ShapeDtypeStruct((B, H, NP1, PC), jnp.float32),
        mesh=mesh,
        compiler_params=pltpu.CompilerParams(use_tc_tiling_on_sc=False,
                                             needs_layout_passes=False),
        scratch_types=[
            pltpu.VMEM((1027 * W,), jnp.int32),
            pltpu.VMEM((H, PC), jnp.float32),
            pltpu.VMEM((H, PC), jnp.float32),
            pltpu.VMEM((N,), jnp.int32),
            pltpu.VMEM((N,), jnp.int32),
            pltpu.VMEM((N * FE,), jnp.int32),
            pltpu.VMEM((N * FE,), jnp.int32),
            pltpu.VMEM((PC,), jnp.float32),
            pltpu.VMEM((PC,), jnp.float32),
            pltpu.VMEM((H,), jnp.float32),
            pltpu.VMEM((H, 16), jnp.float32),
            pltpu.SemaphoreType.DMA,
            pltpu.SemaphoreType.DMA,
            pltpu.SemaphoreType.DMA,
            pltpu.SemaphoreType.DMA,
        ],
    )


def kernel(attn_bias, spatial_pos, attn_edge_type, spatial_W, edge_W, token_W):
    B, NP1, _ = attn_bias.shape
    N = NP1 - 1
    H = spatial_W.shape[1]
    FE = attn_edge_type.shape[-1]
    PC = 136

    tb = jnp.concatenate([spatial_W, edge_W / FE], axis=0).astype(jnp.bfloat16)
    tw = lax.bitcast_convert_type(tb.reshape(-1, H // 2, 2),
                                  jnp.int32).reshape(-1)
    tok = token_W.reshape(H).astype(jnp.float32)
    tokr = jnp.tile(tok.reshape(H, 1), (1, 16))
    sp = spatial_pos.astype(jnp.int32)
    et = attn_edge_type.reshape(B, N, N * FE).astype(jnp.int32)
    abp = jnp.pad(attn_bias, ((0, 0), (0, 0), (0, PC - NP1)))

    run = _bias_encoder_sc(B, N, H, FE)
    outp = run(abp, sp, et, tw, tok, tokr)
    return outp[:, :, :, :NP1]


# speedup vs baseline: 30.8842x; 1.0654x over previous
import jax
import jax.numpy as jnp
from jax import lax
from jax.experimental import pallas as pl
from jax.experimental.pallas import tpu as pltpu
from jax.experimental.pallas import tpu_sc as plsc


def _bias_encoder_sc(B, N, H, FE):
    NP1 = N + 1
    PC = 136
    W = H // 2
    JB = N // 16
    EOFF = 513 * W

    mesh = plsc.VectorSubcoreMesh(core_axis_name="c", subcore_axis_name="s")

    def tec(ab, sp, et, tw, tok, tokr, out,
            tv, t0, t1, spa, spb, eta, etb, aba, abb, tokv, tokrv,
            sia, sib, st0, st1):
        cid = lax.axis_index("c")
        sid = lax.axis_index("s")
        wid = sid * 2 + cid
        pltpu.sync_copy(tw, tv)
        pltpu.sync_copy(tok, tokv)
        pltpu.sync_copy(tokr, tokrv)
        i16 = lax.iota(jnp.int32, 16)
        i4 = i16 * FE
        zero16 = jnp.zeros((16,), jnp.int32)
        lane0 = i16 == 0

        def start_in(b, i, spx, etx, abx, sem):
            pltpu.async_copy(sp.at[b, i, :], spx, sem)
            pltpu.async_copy(et.at[b, i, :], etx, sem)
            pltpu.async_copy(ab.at[b, i + 1, :], abx, sem)

        def wait_in(b, i, spx, etx, abx, sem):
            pltpu.make_async_copy(sp.at[b, i, :], spx, sem).wait()
            pltpu.make_async_copy(et.at[b, i, :], etx, sem).wait()
            pltpu.make_async_copy(ab.at[b, i + 1, :], abx, sem).wait()

        def emit_main(spx, etx, abx, tile):
            av = abx[pl.ds(0, 16)]
            ab0 = jnp.sum(jnp.where(lane0, av, 0.0)) * 2.0
            plsc.store_scatter(tile, [i16, zero16], tokv[pl.ds(0, 16)] + ab0)
            plsc.store_scatter(tile, [i16 + 16, zero16],
                               tokv[pl.ds(16, 16)] + ab0)

            def do_jb(jb):
                ab2 = plsc.load_gather(abx, [i16 + (16 * jb + 1)]) * 2.0
                sp16 = plsc.load_gather(spx, [i16 + 16 * jb])
                spf = sp16 * W
                ef = [plsc.load_gather(etx, [i16 + (f * N + 16 * jb)]) * W
                      + EOFF for f in range(FE)]
                cidx = i16 + (16 * jb + 1)
                for w in range(W):
                    wv = (i16 + w) & (W - 1)
                    gs = [plsc.bitcast(plsc.load_gather(tv, [base + wv]),
                                       jnp.bfloat16)
                          for base in [spf] + ef]
                    while len(gs) > 1:
                        gs = [a + b for a, b in zip(gs[::2], gs[1::2])] \
                            + gs[-1:] * (len(gs) % 2)
                    si = plsc.bitcast(gs[0], jnp.int32)
                    hE = plsc.bitcast(si << 16, jnp.float32) + ab2
                    hO = plsc.bitcast(si & jnp.int32(-65536), jnp.float32) + ab2
                    rE = wv + wv
                    plsc.store_scatter(tile, [rE, cidx], hE)
                    plsc.store_scatter(tile, [rE + 1, cidx], hO)

            def jb_body(jj, c):
                do_jb(2 * jj)
                do_jb(2 * jj + 1)
                return c

            lax.fori_loop(0, JB // 2, jb_body, 0)

        def emit_row0(abx, tile):
            def jb_body(jb, c):
                cidx = i16 + 16 * jb
                msk = cidx < NP1
                ab2 = plsc.load_gather(abx, [jnp.minimum(cidx, PC - 1)]) * 2.0
                for w in range(H):
                    plsc.store_scatter(
                        tile, [jnp.full((16,), w, jnp.int32), cidx],
                        ab2 + tokrv[w, :], mask=msk)
                return c

            lax.fori_loop(0, (PC + 15) // 16, jb_body, 0)

        for bb in range(2):
            b = wid * 2 + bb
            start_in(b, 0, spa, eta, aba, sia)

            def pair(q, c):
                i0 = 2 * q
                wait_in(b, i0, spa, eta, aba, sia)
                start_in(b, i0 + 1, spb, etb, abb, sib)

                @pl.when(q > 0)
                def _():
                    pltpu.make_async_copy(t0, out.at[b, :, i0 + 1, :],
                                          st0).wait()

                emit_main(spa, eta, aba, t0)
                pltpu.async_copy(t0, out.at[b, :, i0 + 1, :], st0)

                wait_in(b, i0 + 1, spb, etb, abb, sib)

                @pl.when(q < N // 2 - 1)
                def _():
                    start_in(b, i0 + 2, spa, eta, aba, sia)

                @pl.when(q > 0)
                def _():
                    pltpu.make_async_copy(t1, out.at[b, :, i0 + 2, :],
                                          st1).wait()

                emit_main(spb, etb, abb, t1)
                pltpu.async_copy(t1, out.at[b, :, i0 + 2, :], st1)
                return c

            lax.fori_loop(0, N // 2, pair, 0)
            pltpu.make_async_copy(t0, out.at[b, :, N - 1, :], st0).wait()
            pltpu.make_async_copy(t1, out.at[b, :, N, :], st1).wait()
            pltpu.sync_copy(ab.at[b, 0, :], aba)
            emit_row0(aba, t0)
            pltpu.sync_copy(t0, out.at[b, :, 0, :])

    return pl.kernel(
        tec,
        out_type=jax.ShapeDtypeStruct((B, H, NP1, PC), jnp.float32),
        mesh=mesh,
        compiler_params=pltpu.CompilerParams(use_tc_tiling_on_sc=False,
                                             needs_layout_passes=False),
        scratch_types=[
            pltpu.VMEM((1027 * W,), jnp.int32),
            pltpu.VMEM((H, PC), jnp.float32),
            pltpu.VMEM((H, PC), jnp.float32),
            pltpu.VMEM((N,), jnp.int32),
            pltpu.VMEM((N,), jnp.int32),
            pltpu.VMEM((N * FE,), jnp.int32),
            pltpu.VMEM((N * FE,), jnp.int32),
            pltpu.VMEM((PC,), jnp.float32),
            pltpu.VMEM((PC,), jnp.float32),
            pltpu.VMEM((H,), jnp.float32),
            pltpu.VMEM((H, 16), jnp.float32),
            pltpu.SemaphoreType.DMA,
            pltpu.SemaphoreType.DMA,
            pltpu.SemaphoreType.DMA,
            pltpu.SemaphoreType.DMA,
        ],
    )


def kernel(attn_bias, spatial_pos, attn_edge_type, spatial_W, edge_W, token_W):
    B, NP1, _ = attn_bias.shape
    N = NP1 - 1
    H = spatial_W.shape[1]
    FE = attn_edge_type.shape[-1]
    PC = 136

    tb = jnp.concatenate([spatial_W, edge_W / FE], axis=0).astype(jnp.bfloat16)
    tw = lax.bitcast_convert_type(tb.reshape(-1, H // 2, 2),
                                  jnp.int32).reshape(-1)
    tok = token_W.reshape(H).astype(jnp.float32)
    tokr = jnp.tile(tok.reshape(H, 1), (1, 16))
    sp = spatial_pos.astype(jnp.int32)
    et = attn_edge_type.transpose(0, 1, 3, 2).reshape(B, N, FE * N)
    et = et.astype(jnp.int32)
    abp = jnp.pad(attn_bias, ((0, 0), (0, 0), (0, PC - NP1)))

    run = _bias_encoder_sc(B, N, H, FE)
    outp = run(abp, sp, et, tw, tok, tokr)
    return outp[:, :, :, :NP1]


# speedup vs baseline: 30.8966x; 1.0004x over previous
import jax
import jax.numpy as jnp
from jax import lax
from jax.experimental import pallas as pl
from jax.experimental.pallas import tpu as pltpu
from jax.experimental.pallas import tpu_sc as plsc


def _bias_encoder_sc(B, N, H, FE):
    NP1 = N + 1
    PC = 136
    W = H // 2
    JB = N // 16
    EOFF = 513 * W

    mesh = plsc.VectorSubcoreMesh(core_axis_name="c", subcore_axis_name="s")

    def tec(ab, sp, et, tw, tok, tokr, out,
            tv, t0, t1, spa, spb, eta, etb, aba, abb, tokv, tokrv,
            sia, sib, st0, st1):
        cid = lax.axis_index("c")
        sid = lax.axis_index("s")
        wid = sid * 2 + cid
        pltpu.sync_copy(tw, tv)
        pltpu.sync_copy(tok, tokv)
        pltpu.sync_copy(tokr, tokrv)
        i16 = lax.iota(jnp.int32, 16)
        i4 = i16 * FE
        zero16 = jnp.zeros((16,), jnp.int32)
        lane0 = i16 == 0

        def start_in(b, i, spx, etx, abx, sem):
            pltpu.async_copy(sp.at[b, i, :], spx, sem)
            pltpu.async_copy(et.at[b, i, :], etx, sem)
            pltpu.async_copy(ab.at[b, i + 1, :], abx, sem)

        def wait_in(b, i, spx, etx, abx, sem):
            pltpu.make_async_copy(sp.at[b, i, :], spx, sem).wait()
            pltpu.make_async_copy(et.at[b, i, :], etx, sem).wait()
            pltpu.make_async_copy(ab.at[b, i + 1, :], abx, sem).wait()

        def emit_main(spx, etx, abx, tile):
            av = abx[pl.ds(0, 16)]
            ab0 = jnp.sum(jnp.where(lane0, av, 0.0)) * 2.0
            plsc.store_scatter(tile, [i16, zero16], tokv[pl.ds(0, 16)] + ab0)
            plsc.store_scatter(tile, [i16 + 16, zero16],
                               tokv[pl.ds(16, 16)] + ab0)

            def do_jb(jb):
                ab2 = plsc.load_gather(abx, [i16 + (16 * jb + 1)]) * 2.0
                sp16 = plsc.load_gather(spx, [i16 + 16 * jb])
                spf = sp16 * W
                ef = [plsc.load_gather(etx, [i16 + (f * N + 16 * jb)]) * W
                      + EOFF for f in range(FE)]
                cidx = i16 + (16 * jb + 1)
                for w in range(W):
                    wv = (i16 + w) & (W - 1)
                    gs = [plsc.bitcast(plsc.load_gather(tv, [base + wv]),
                                       jnp.bfloat16)
                          for base in [spf] + ef]
                    while len(gs) > 1:
                        gs = [a + b for a, b in zip(gs[::2], gs[1::2])] \
                            + gs[-1:] * (len(gs) % 2)
                    si = plsc.bitcast(gs[0], jnp.int32)
                    hE = plsc.bitcast(si << 16, jnp.float32) + ab2
                    hO = plsc.bitcast(si & jnp.int32(-65536), jnp.float32) + ab2
                    rE = wv + wv
                    plsc.store_scatter(tile, [rE, cidx], hE)
                    plsc.store_scatter(tile, [rE + 1, cidx], hO)

            def jb_body(jj, c):
                do_jb(2 * jj)
                do_jb(2 * jj + 1)
                return c

            lax.fori_loop(0, JB // 2, jb_body, 0)

        def emit_row0(abx, tile):
            def jb_body(jb, c):
                cidx = i16 + 16 * jb
                msk = cidx < NP1
                ab2 = plsc.load_gather(abx, [jnp.minimum(cidx, PC - 1)]) * 2.0
                for w in range(H):
                    plsc.store_scatter(
                        tile, [jnp.full((16,), w, jnp.int32), cidx],
                        ab2 + tokrv[w, :], mask=msk)
                return c

            lax.fori_loop(0, (PC + 15) // 16, jb_body, 0)

        for bb in range(2):
            b = wid * 2 + bb
            start_in(b, 0, spa, eta, aba, sia)

            def pair(q, c):
                i0 = 2 * q
                wait_in(b, i0, spa, eta, aba, sia)
                start_in(b, i0 + 1, spb, etb, abb, sib)

                @pl.when(q > 0)
                def _():
                    pltpu.make_async_copy(t0, out.at[b, :, i0 + 1, :],
                                          st0).wait()

                emit_main(spa, eta, aba, t0)
                pltpu.async_copy(t0, out.at[b, :, i0 + 1, :], st0)

                wait_in(b, i0 + 1, spb, etb, abb, sib)

                @pl.when(q < N // 2 - 1)
                def _():
                    start_in(b, i0 + 2, spa, eta, aba, sia)

                @pl.when(q > 0)
                def _():
                    pltpu.make_async_copy(t1, out.at[b, :, i0 + 2, :],
                                          st1).wait()

                emit_main(spb, etb, abb, t1)
                pltpu.async_copy(t1, out.at[b, :, i0 + 2, :], st1)
                return c

            lax.fori_loop(0, N // 2, pair, 0)
            pltpu.make_async_copy(t0, out.at[b, :, N - 1, :], st0).wait()
            pltpu.make_async_copy(t1, out.at[b, :, N, :], st1).wait()
            pltpu.sync_copy(ab.at[b, 0, :], aba)
            emit_row0(aba, t0)
            pltpu.sync_copy(t0, out.at[b, :, 0, :])

    return pl.kernel(
        tec,
        out_type=jax.ShapeDtypeStruct((B, H, NP1, NP1), jnp.float32),
        mesh=mesh,
        compiler_params=pltpu.CompilerParams(use_tc_tiling_on_sc=False,
                                             needs_layout_passes=False),
        scratch_types=[
            pltpu.VMEM((1027 * W,), jnp.int32),
            pltpu.VMEM((H, NP1), jnp.float32),
            pltpu.VMEM((H, NP1), jnp.float32),
            pltpu.VMEM((N,), jnp.int32),
            pltpu.VMEM((N,), jnp.int32),
            pltpu.VMEM((N * FE,), jnp.int32),
            pltpu.VMEM((N * FE,), jnp.int32),
            pltpu.VMEM((PC,), jnp.float32),
            pltpu.VMEM((PC,), jnp.float32),
            pltpu.VMEM((H,), jnp.float32),
            pltpu.VMEM((H, 16), jnp.float32),
            pltpu.SemaphoreType.DMA,
            pltpu.SemaphoreType.DMA,
            pltpu.SemaphoreType.DMA,
            pltpu.SemaphoreType.DMA,
        ],
    )


def kernel(attn_bias, spatial_pos, attn_edge_type, spatial_W, edge_W, token_W):
    B, NP1, _ = attn_bias.shape
    N = NP1 - 1
    H = spatial_W.shape[1]
    FE = attn_edge_type.shape[-1]
    PC = 136

    tb = jnp.concatenate([spatial_W, edge_W / FE], axis=0).astype(jnp.bfloat16)
    tw = lax.bitcast_convert_type(tb.reshape(-1, H // 2, 2),
                                  jnp.int32).reshape(-1)
    tok = token_W.reshape(H).astype(jnp.float32)
    tokr = jnp.tile(tok.reshape(H, 1), (1, 16))
    sp = spatial_pos.astype(jnp.int32)
    et = attn_edge_type.transpose(0, 1, 3, 2).reshape(B, N, FE * N)
    et = et.astype(jnp.int32)
    abp = jnp.pad(attn_bias, ((0, 0), (0, 0), (0, PC - NP1)))

    run = _bias_encoder_sc(B, N, H, FE)
    return run(abp, sp, et, tw, tok, tokr)
